# baseline (speedup 1.0000x reference)
import functools
import math

import jax
import jax.numpy as jnp
from jax import lax
from jax.experimental import pallas as pl
from jax.experimental.pallas import tpu as pltpu
from jax.experimental.pallas import tpu_sc as plsc

K = 20
N = 1024
B = 16
RBLK = 512
PBLK = 128
WINDOW = 128
GBATCH = 4
CHAIN = 8
NEG = -3.0e38


def _bmm(a, b):
    return jnp.dot(a.astype(jnp.bfloat16), b.astype(jnp.bfloat16),
                   preferred_element_type=jnp.float32)


def _topk_body(frows_ref, fcols_ref, idx_ref, d_ref):
    b = pl.program_id(0)
    f = frows_ref[0]
    fc = fcols_ref[0]
    inner = _bmm(f, fc)
    xxr = jnp.sum(f * f, axis=1, keepdims=True)
    xxc = jnp.sum(fc * fc, axis=0, keepdims=True)
    d_ref[...] = 2.0 * inner - xxr - xxc
    iota = lax.broadcasted_iota(jnp.int32, (N, RBLK), 0)
    base = b * N
    for k2 in range(K // 2):
        d = d_ref[...]
        am1 = jnp.argmax(d, axis=0)[None, :].astype(jnp.int32)
        idx_ref[0, pl.ds(2 * k2, 1), :] = am1 + base
        d = jnp.where(iota == am1, NEG, d)
        am2 = jnp.argmax(d, axis=0)[None, :].astype(jnp.int32)
        idx_ref[0, pl.ds(2 * k2 + 1, 1), :] = am2 + base
        d_ref[...] = jnp.where(iota == am2, NEG, d)


def _topk(frows, fcols):
    nb = frows.shape[0]
    c = frows.shape[2]
    return pl.pallas_call(
        _topk_body,
        grid=(nb, N // RBLK),
        in_specs=[
            pl.BlockSpec((1, N, c), lambda b, i: (b, 0, 0)),
            pl.BlockSpec((1, c, RBLK), lambda b, i: (b, 0, i)),
        ],
        out_specs=pl.BlockSpec((1, K, RBLK), lambda b, i: (b, 0, i)),
        out_shape=jax.ShapeDtypeStruct((nb, K, N), jnp.int32),
        scratch_shapes=[pltpu.VMEM((N, RBLK), jnp.float32)],
    )(frows, fcols)


def _sc_gather(table, idx_flat):
    e_total = idx_flat.shape[0]
    d = table.shape[1]
    nw = 32
    steps = e_total // (GBATCH * WINDOW * nw)
    idx2 = idx_flat.reshape(1, e_total)
    mesh = plsc.VectorSubcoreMesh(core_axis_name="core",
                                  subcore_axis_name="subcore")

    @functools.partial(
        pl.kernel,
        out_type=jax.ShapeDtypeStruct((e_total, d), table.dtype),
        mesh=mesh,
        compiler_params=pltpu.CompilerParams(use_tc_tiling_on_sc=False),
    )
    def gather_kernel(table_hbm, i_hbm, o_hbm):
        def body(i_vmem, o_vmem):
            for j in range(GBATCH):
                pltpu.sync_copy(
                    table_hbm.at[i_vmem.at[0, pl.ds(j * WINDOW, WINDOW)]],
                    o_vmem.at[pl.ds(j * WINDOW, WINDOW)])

        pltpu.emit_pipeline(
            body,
            grid=(nw, steps),
            in_specs=[pl.BlockSpec((1, GBATCH * WINDOW),
                                   index_map=lambda w, i: (0, w * steps + i))],
            out_specs=[pl.BlockSpec((GBATCH * WINDOW, d),
                                    index_map=lambda w, i: (w * steps + i, 0))],
            core_axis_name=("core", "subcore"),
            dimension_semantics=(pltpu.PARALLEL, pltpu.PARALLEL),
        )(i_hbm, o_hbm)

    return gather_kernel(table, idx2)


def _edge2_body(e_ref, f_ref, wd_ref, wc_ref, ba_ref, sga_ref, ea_ref,
                wb_ref, bb_ref, sgb_ref, eb_ref, out_ref):
    c = f_ref.shape[2]
    cen = f_ref[0]
    diff = (e_ref[0][:, :, :c] - cen[None, :, :]).reshape(K * PBLK, c)
    ycb = _bmm(cen, wc_ref[...]) + ba_ref[...]
    y = _bmm(diff, wd_ref[...]).reshape(K, PBLK, 64) + ycb[None]
    h = y * sga_ref[...][None] + ea_ref[...][None]
    h = jnp.maximum(h, 0.2 * h)
    z = _bmm(h.reshape(K * PBLK, 64), wb_ref[...]).reshape(K, PBLK, 64)
    acc = (jnp.max(z, axis=0) + bb_ref[...]) * sgb_ref[...] + eb_ref[...]
    out_ref[0] = jnp.maximum(acc, 0.2 * acc)


def _edge1_body(e_ref, f_ref, wd_ref, wc_ref, ba_ref, sga_ref, ea_ref,
                out_ref):
    c = f_ref.shape[2]
    cen = f_ref[0]
    diff = (e_ref[0][:, :, :c] - cen[None, :, :]).reshape(K * PBLK, c)
    dm = jnp.max(_bmm(diff, wd_ref[...]).reshape(K, PBLK, 64), axis=0)
    y = dm + _bmm(cen, wc_ref[...]) + ba_ref[...]
    yb = y * sga_ref[...] + ea_ref[...]
    out_ref[0] = jnp.maximum(yb, 0.2 * yb)


def _row(v):
    return v.reshape(1, 64)


def _sg(g):
    return _row(g * (1.0 / jnp.sqrt(jnp.float32(1.0 + 1e-5))))


def _edge_conv(edges4, frows, wa, ba, ga, ea, wb=None, bb=None, gb=None,
               eb=None):
    c = frows.shape[2]
    cp = edges4.shape[3]
    w_specs = [pl.BlockSpec((c, 64), lambda b, i: (0, 0))] * 2 + \
              [pl.BlockSpec((1, 64), lambda b, i: (0, 0))] * 3
    specs = [
        pl.BlockSpec((1, K, PBLK, cp), lambda b, i: (b, 0, i, 0)),
        pl.BlockSpec((1, PBLK, c), lambda b, i: (b, i, 0)),
    ] + w_specs
    args = [edges4, frows, wa[:, :c].T, wa[:, c:].T, _row(ba), _sg(ga),
            _row(ea)]
    body = _edge1_body
    if wb is not None:
        specs += [pl.BlockSpec((64, 64), lambda b, i: (0, 0))] + \
                 [pl.BlockSpec((1, 64), lambda b, i: (0, 0))] * 3
        args += [wb.T, _row(bb), _sg(gb), _row(eb)]
        body = _edge2_body
    return pl.pallas_call(
        body,
        grid=(edges4.shape[0], N // PBLK),
        in_specs=specs,
        out_specs=pl.BlockSpec((1, PBLK, 64), lambda b, i: (b, i, 0)),
        out_shape=jax.ShapeDtypeStruct((edges4.shape[0], N, 64), jnp.float32),
    )(*args)


def kernel(x, W0a, b0a, g0a, e0a, W0b, b0b, g0b, e0b,
           W1a, b1a, g1a, e1a, W1b, b1b, g1b, e1b,
           W2, b2, g2, e2, W3, b3, g3, e3):

    def stage(frows, fcols, table):
        idx = _topk(frows, fcols)
        edges = _sc_gather(table, idx.reshape(-1))
        return edges.reshape(frows.shape[0], K, N, table.shape[1])

    def chain(xc):
        nb = xc.shape[0]
        f0r = jnp.transpose(xc, (0, 2, 1))
        t0 = jnp.pad(f0r, ((0, 0), (0, 0), (0, 10))).reshape(nb * N, 16)
        e1 = stage(f0r, xc, t0)
        x1r = _edge_conv(e1, f0r, W0a, b0a, g0a, e0a, W0b, b0b, g0b, e0b)

        x1c = jnp.transpose(x1r, (0, 2, 1))
        e2s = stage(x1r, x1c, x1r.reshape(nb * N, 64))
        x2r = _edge_conv(e2s, x1r, W1a, b1a, g1a, e1a, W1b, b1b, g1b, e1b)

        x2c = jnp.transpose(x2r, (0, 2, 1))
        e3s = stage(x2r, x2c, x2r.reshape(nb * N, 64))
        x3r = _edge_conv(e3s, x2r, W2, b2, g2, e2)

        x3c = jnp.transpose(x3r, (0, 2, 1))
        e4s = stage(x3r, x3c, x3r.reshape(nb * N, 64))
        x4r = _edge_conv(e4s, x3r, W3, b3, g3, e3)

        return jnp.concatenate(
            [x1c, x2c, x3c, jnp.transpose(x4r, (0, 2, 1))], axis=1)

    nc = B // CHAIN
    outs = [chain(x[i * CHAIN:(i + 1) * CHAIN]) for i in range(nc)]
    return jnp.concatenate(outs, axis=0)


# speedup vs baseline: 1.0731x; 1.0731x over previous
import functools
import math

import jax
import jax.numpy as jnp
from jax import lax
from jax.experimental import pallas as pl
from jax.experimental.pallas import tpu as pltpu
from jax.experimental.pallas import tpu_sc as plsc

K = 20
N = 1024
B = 16
RBLK = 256
PBLK = 256
WINDOW = 128
GBATCH = 4
CHAIN = 8
NEG = -3.0e38


def _bmm(a, b):
    return jnp.dot(a.astype(jnp.bfloat16), b.astype(jnp.bfloat16),
                   preferred_element_type=jnp.float32)


def _topk_body(frows_ref, fcols_ref, idx_ref, d_ref):
    b = pl.program_id(0)
    f = frows_ref[0]
    fc = fcols_ref[0]
    inner = _bmm(f, fc)
    xxr = jnp.sum(f * f, axis=1, keepdims=True)
    xxc = jnp.sum(fc * fc, axis=0, keepdims=True)
    d_ref[...] = 2.0 * inner - xxr - xxc
    iota = lax.broadcasted_iota(jnp.int32, (N, RBLK), 0)
    base = b * N
    for k2 in range(K // 2):
        d = d_ref[...]
        am1 = jnp.argmax(d, axis=0)[None, :].astype(jnp.int32)
        idx_ref[0, pl.ds(2 * k2, 1), :] = am1 + base
        d = jnp.where(iota == am1, NEG, d)
        am2 = jnp.argmax(d, axis=0)[None, :].astype(jnp.int32)
        idx_ref[0, pl.ds(2 * k2 + 1, 1), :] = am2 + base
        d_ref[...] = jnp.where(iota == am2, NEG, d)


def _topk(frows, fcols):
    nb = frows.shape[0]
    c = frows.shape[2]
    return pl.pallas_call(
        _topk_body,
        grid=(nb, N // RBLK),
        in_specs=[
            pl.BlockSpec((1, N, c), lambda b, i: (b, 0, 0)),
            pl.BlockSpec((1, c, RBLK), lambda b, i: (b, 0, i)),
        ],
        out_specs=pl.BlockSpec((1, K, RBLK), lambda b, i: (b, 0, i)),
        out_shape=jax.ShapeDtypeStruct((nb, K, N), jnp.int32),
        scratch_shapes=[pltpu.VMEM((N, RBLK), jnp.float32)],
    )(frows, fcols)


def _sc_gather(table, idx_flat):
    e_total = idx_flat.shape[0]
    d = table.shape[1]
    nw = 32
    steps = e_total // (GBATCH * WINDOW * nw)
    idx2 = idx_flat.reshape(1, e_total)
    mesh = plsc.VectorSubcoreMesh(core_axis_name="core",
                                  subcore_axis_name="subcore")

    @functools.partial(
        pl.kernel,
        out_type=jax.ShapeDtypeStruct((e_total, d), table.dtype),
        mesh=mesh,
        compiler_params=pltpu.CompilerParams(use_tc_tiling_on_sc=False),
    )
    def gather_kernel(table_hbm, i_hbm, o_hbm):
        def body(i_vmem, o_vmem):
            for j in range(GBATCH):
                pltpu.sync_copy(
                    table_hbm.at[i_vmem.at[0, pl.ds(j * WINDOW, WINDOW)]],
                    o_vmem.at[pl.ds(j * WINDOW, WINDOW)])

        pltpu.emit_pipeline(
            body,
            grid=(nw, steps),
            in_specs=[pl.BlockSpec((1, GBATCH * WINDOW),
                                   index_map=lambda w, i: (0, w * steps + i))],
            out_specs=[pl.BlockSpec((GBATCH * WINDOW, d),
                                    index_map=lambda w, i: (w * steps + i, 0))],
            core_axis_name=("core", "subcore"),
            dimension_semantics=(pltpu.PARALLEL, pltpu.PARALLEL),
        )(i_hbm, o_hbm)

    return gather_kernel(table, idx2)


def _edge2_body(e_ref, f_ref, wd_ref, wc_ref, ba_ref, sga_ref, ea_ref,
                wb_ref, bb_ref, sgb_ref, eb_ref, out_ref):
    c = f_ref.shape[2]
    cen = f_ref[0]
    diff = (e_ref[0][:, :, :c] - cen[None, :, :]).reshape(K * PBLK, c)
    ycb = _bmm(cen, wc_ref[...]) + ba_ref[...]
    y = _bmm(diff, wd_ref[...]).reshape(K, PBLK, 64) + ycb[None]
    h = y * sga_ref[...][None] + ea_ref[...][None]
    h = jnp.maximum(h, 0.2 * h)
    z = _bmm(h.reshape(K * PBLK, 64), wb_ref[...]).reshape(K, PBLK, 64)
    acc = (jnp.max(z, axis=0) + bb_ref[...]) * sgb_ref[...] + eb_ref[...]
    out_ref[0] = jnp.maximum(acc, 0.2 * acc)


def _edge1_body(e_ref, f_ref, wd_ref, wc_ref, ba_ref, sga_ref, ea_ref,
                out_ref):
    c = f_ref.shape[2]
    cen = f_ref[0]
    diff = (e_ref[0][:, :, :c] - cen[None, :, :]).reshape(K * PBLK, c)
    dm = jnp.max(_bmm(diff, wd_ref[...]).reshape(K, PBLK, 64), axis=0)
    y = dm + _bmm(cen, wc_ref[...]) + ba_ref[...]
    yb = y * sga_ref[...] + ea_ref[...]
    out_ref[0] = jnp.maximum(yb, 0.2 * yb)


def _row(v):
    return v.reshape(1, 64)


def _sg(g):
    return _row(g * (1.0 / jnp.sqrt(jnp.float32(1.0 + 1e-5))))


def _edge_conv(edges4, frows, wa, ba, ga, ea, wb=None, bb=None, gb=None,
               eb=None):
    c = frows.shape[2]
    cp = edges4.shape[3]
    w_specs = [pl.BlockSpec((c, 64), lambda b, i: (0, 0))] * 2 + \
              [pl.BlockSpec((1, 64), lambda b, i: (0, 0))] * 3
    specs = [
        pl.BlockSpec((1, K, PBLK, cp), lambda b, i: (b, 0, i, 0)),
        pl.BlockSpec((1, PBLK, c), lambda b, i: (b, i, 0)),
    ] + w_specs
    args = [edges4, frows, wa[:, :c].T, wa[:, c:].T, _row(ba), _sg(ga),
            _row(ea)]
    body = _edge1_body
    if wb is not None:
        specs += [pl.BlockSpec((64, 64), lambda b, i: (0, 0))] + \
                 [pl.BlockSpec((1, 64), lambda b, i: (0, 0))] * 3
        args += [wb.T, _row(bb), _sg(gb), _row(eb)]
        body = _edge2_body
    return pl.pallas_call(
        body,
        grid=(edges4.shape[0], N // PBLK),
        in_specs=specs,
        out_specs=pl.BlockSpec((1, PBLK, 64), lambda b, i: (b, i, 0)),
        out_shape=jax.ShapeDtypeStruct((edges4.shape[0], N, 64), jnp.float32),
    )(*args)


def kernel(x, W0a, b0a, g0a, e0a, W0b, b0b, g0b, e0b,
           W1a, b1a, g1a, e1a, W1b, b1b, g1b, e1b,
           W2, b2, g2, e2, W3, b3, g3, e3):

    def stage(frows, fcols, table):
        idx = _topk(frows, fcols)
        edges = _sc_gather(table, idx.reshape(-1))
        return edges.reshape(frows.shape[0], K, N, table.shape[1])

    def chain(xc):
        nb = xc.shape[0]
        f0r = jnp.transpose(xc, (0, 2, 1))
        t0 = jnp.pad(f0r, ((0, 0), (0, 0), (0, 10))).reshape(nb * N, 16)
        e1 = stage(f0r, xc, t0)
        x1r = _edge_conv(e1, f0r, W0a, b0a, g0a, e0a, W0b, b0b, g0b, e0b)

        x1c = jnp.transpose(x1r, (0, 2, 1))
        e2s = stage(x1r, x1c, x1r.reshape(nb * N, 64))
        x2r = _edge_conv(e2s, x1r, W1a, b1a, g1a, e1a, W1b, b1b, g1b, e1b)

        x2c = jnp.transpose(x2r, (0, 2, 1))
        e3s = stage(x2r, x2c, x2r.reshape(nb * N, 64))
        x3r = _edge_conv(e3s, x2r, W2, b2, g2, e2)

        x3c = jnp.transpose(x3r, (0, 2, 1))
        e4s = stage(x3r, x3c, x3r.reshape(nb * N, 64))
        x4r = _edge_conv(e4s, x3r, W3, b3, g3, e3)

        return jnp.concatenate(
            [x1c, x2c, x3c, jnp.transpose(x4r, (0, 2, 1))], axis=1)

    nc = B // CHAIN
    outs = [chain(x[i * CHAIN:(i + 1) * CHAIN]) for i in range(nc)]
    return jnp.concatenate(outs, axis=0)


# speedup vs baseline: 1.1113x; 1.0356x over previous
import functools
import math

import jax
import jax.numpy as jnp
from jax import lax
from jax.experimental import pallas as pl
from jax.experimental.pallas import tpu as pltpu
from jax.experimental.pallas import tpu_sc as plsc

K = 20
N = 1024
B = 16
RBLK = 256
PBLK = 512
WINDOW = 128
GBATCH = 4
CHAIN = 8
NEG = -3.0e38


def _bmm(a, b):
    return jnp.dot(a.astype(jnp.bfloat16), b.astype(jnp.bfloat16),
                   preferred_element_type=jnp.float32)


def _topk_body(frows_ref, fcols_ref, idx_ref, d_ref):
    b = pl.program_id(0)
    f = frows_ref[0]
    fc = fcols_ref[0]
    inner = _bmm(f, fc)
    xxr = jnp.sum(f * f, axis=1, keepdims=True)
    xxc = jnp.sum(fc * fc, axis=0, keepdims=True)
    d_ref[...] = 2.0 * inner - xxr - xxc
    iota = lax.broadcasted_iota(jnp.int32, (N, RBLK), 0)
    base = b * N
    for k2 in range(K // 2):
        d = d_ref[...]
        am1 = jnp.argmax(d, axis=0)[None, :].astype(jnp.int32)
        idx_ref[0, pl.ds(2 * k2, 1), :] = am1 + base
        d = jnp.where(iota == am1, NEG, d)
        am2 = jnp.argmax(d, axis=0)[None, :].astype(jnp.int32)
        idx_ref[0, pl.ds(2 * k2 + 1, 1), :] = am2 + base
        d_ref[...] = jnp.where(iota == am2, NEG, d)


def _topk(frows, fcols):
    nb = frows.shape[0]
    c = frows.shape[2]
    return pl.pallas_call(
        _topk_body,
        grid=(nb, N // RBLK),
        in_specs=[
            pl.BlockSpec((1, N, c), lambda b, i: (b, 0, 0)),
            pl.BlockSpec((1, c, RBLK), lambda b, i: (b, 0, i)),
        ],
        out_specs=pl.BlockSpec((1, K, RBLK), lambda b, i: (b, 0, i)),
        out_shape=jax.ShapeDtypeStruct((nb, K, N), jnp.int32),
        scratch_shapes=[pltpu.VMEM((N, RBLK), jnp.float32)],
    )(frows, fcols)


def _sc_gather(table, idx_flat):
    e_total = idx_flat.shape[0]
    d = table.shape[1]
    nw = 32
    steps = e_total // (GBATCH * WINDOW * nw)
    idx2 = idx_flat.reshape(1, e_total)
    mesh = plsc.VectorSubcoreMesh(core_axis_name="core",
                                  subcore_axis_name="subcore")

    @functools.partial(
        pl.kernel,
        out_type=jax.ShapeDtypeStruct((e_total, d), table.dtype),
        mesh=mesh,
        compiler_params=pltpu.CompilerParams(use_tc_tiling_on_sc=False),
    )
    def gather_kernel(table_hbm, i_hbm, o_hbm):
        def body(i_vmem, o_vmem):
            for j in range(GBATCH):
                pltpu.sync_copy(
                    table_hbm.at[i_vmem.at[0, pl.ds(j * WINDOW, WINDOW)]],
                    o_vmem.at[pl.ds(j * WINDOW, WINDOW)])

        pltpu.emit_pipeline(
            body,
            grid=(nw, steps),
            in_specs=[pl.BlockSpec((1, GBATCH * WINDOW),
                                   index_map=lambda w, i: (0, w * steps + i))],
            out_specs=[pl.BlockSpec((GBATCH * WINDOW, d),
                                    index_map=lambda w, i: (w * steps + i, 0))],
            core_axis_name=("core", "subcore"),
            dimension_semantics=(pltpu.PARALLEL, pltpu.PARALLEL),
        )(i_hbm, o_hbm)

    return gather_kernel(table, idx2)


def _edge2_body(e_ref, f_ref, wd_ref, wc_ref, ba_ref, sga_ref, ea_ref,
                wb_ref, bb_ref, sgb_ref, eb_ref, out_ref):
    c = f_ref.shape[2]
    cen = f_ref[0]
    diff = (e_ref[0][:, :, :c] - cen[None, :, :]).reshape(K * PBLK, c)
    ycb = _bmm(cen, wc_ref[...]) + ba_ref[...]
    y = _bmm(diff, wd_ref[...]).reshape(K, PBLK, 64) + ycb[None]
    h = y * sga_ref[...][None] + ea_ref[...][None]
    h = jnp.maximum(h, 0.2 * h)
    z = _bmm(h.reshape(K * PBLK, 64), wb_ref[...]).reshape(K, PBLK, 64)
    acc = (jnp.max(z, axis=0) + bb_ref[...]) * sgb_ref[...] + eb_ref[...]
    out_ref[0] = jnp.maximum(acc, 0.2 * acc)


def _edge1_body(e_ref, f_ref, wd_ref, wc_ref, ba_ref, sga_ref, ea_ref,
                out_ref):
    c = f_ref.shape[2]
    cen = f_ref[0]
    diff = (e_ref[0][:, :, :c] - cen[None, :, :]).reshape(K * PBLK, c)
    dm = jnp.max(_bmm(diff, wd_ref[...]).reshape(K, PBLK, 64), axis=0)
    y = dm + _bmm(cen, wc_ref[...]) + ba_ref[...]
    yb = y * sga_ref[...] + ea_ref[...]
    out_ref[0] = jnp.maximum(yb, 0.2 * yb)


def _row(v):
    return v.reshape(1, 64)


def _sg(g):
    return _row(g * (1.0 / jnp.sqrt(jnp.float32(1.0 + 1e-5))))


def _edge_conv(edges4, frows, wa, ba, ga, ea, wb=None, bb=None, gb=None,
               eb=None):
    c = frows.shape[2]
    cp = edges4.shape[3]
    w_specs = [pl.BlockSpec((c, 64), lambda b, i: (0, 0))] * 2 + \
              [pl.BlockSpec((1, 64), lambda b, i: (0, 0))] * 3
    specs = [
        pl.BlockSpec((1, K, PBLK, cp), lambda b, i: (b, 0, i, 0)),
        pl.BlockSpec((1, PBLK, c), lambda b, i: (b, i, 0)),
    ] + w_specs
    args = [edges4, frows, wa[:, :c].T, wa[:, c:].T, _row(ba), _sg(ga),
            _row(ea)]
    body = _edge1_body
    if wb is not None:
        specs += [pl.BlockSpec((64, 64), lambda b, i: (0, 0))] + \
                 [pl.BlockSpec((1, 64), lambda b, i: (0, 0))] * 3
        args += [wb.T, _row(bb), _sg(gb), _row(eb)]
        body = _edge2_body
    return pl.pallas_call(
        body,
        grid=(edges4.shape[0], N // PBLK),
        in_specs=specs,
        out_specs=pl.BlockSpec((1, PBLK, 64), lambda b, i: (b, i, 0)),
        out_shape=jax.ShapeDtypeStruct((edges4.shape[0], N, 64), jnp.float32),
    )(*args)


def kernel(x, W0a, b0a, g0a, e0a, W0b, b0b, g0b, e0b,
           W1a, b1a, g1a, e1a, W1b, b1b, g1b, e1b,
           W2, b2, g2, e2, W3, b3, g3, e3):

    def stage(frows, fcols, table):
        idx = _topk(frows, fcols)
        edges = _sc_gather(table, idx.reshape(-1))
        return edges.reshape(frows.shape[0], K, N, table.shape[1])

    def chain(xc):
        nb = xc.shape[0]
        f0r = jnp.transpose(xc, (0, 2, 1))
        t0 = jnp.pad(f0r, ((0, 0), (0, 0), (0, 10))).reshape(nb * N, 16)
        e1 = stage(f0r, xc, t0)
        x1r = _edge_conv(e1, f0r, W0a, b0a, g0a, e0a, W0b, b0b, g0b, e0b)

        x1c = jnp.transpose(x1r, (0, 2, 1))
        e2s = stage(x1r, x1c, x1r.reshape(nb * N, 64))
        x2r = _edge_conv(e2s, x1r, W1a, b1a, g1a, e1a, W1b, b1b, g1b, e1b)

        x2c = jnp.transpose(x2r, (0, 2, 1))
        e3s = stage(x2r, x2c, x2r.reshape(nb * N, 64))
        x3r = _edge_conv(e3s, x2r, W2, b2, g2, e2)

        x3c = jnp.transpose(x3r, (0, 2, 1))
        e4s = stage(x3r, x3c, x3r.reshape(nb * N, 64))
        x4r = _edge_conv(e4s, x3r, W3, b3, g3, e3)

        return jnp.concatenate(
            [x1c, x2c, x3c, jnp.transpose(x4r, (0, 2, 1))], axis=1)

    nc = B // CHAIN
    outs = [chain(x[i * CHAIN:(i + 1) * CHAIN]) for i in range(nc)]
    return jnp.concatenate(outs, axis=0)
